# baseline (speedup 1.0000x reference)
import dataclasses
import functools

import jax
import jax.numpy as jnp
from jax import lax
from jax.experimental import pallas as pl
from jax.experimental.pallas import tpu as pltpu
from jax.experimental.pallas import tpu_sc as plsc

_B = 16384
_V = 1024
_D = 128
_CUR = 5
_L = 16
_NC = 2
_NS = 16
_NW = _NC * _NS
_RPW = _B // _NW
_G = _RPW // _L
_CH = 128
_NCHUNK = _RPW // _CH
_BPR = _V // _L


def _sc_search(ls_v, us_v, ip_v):
    mesh = plsc.VectorSubcoreMesh(
        core_axis_name="c", subcore_axis_name="s", num_cores=_NC,
        num_subcores=_NS)
    cp = pltpu.CompilerParams()
    for field, val in (("needs_layout_passes", False),
                       ("use_tc_tiling_on_sc", False)):
        if field in pltpu.CompilerParams.__dataclass_fields__:
            cp = dataclasses.replace(cp, **{field: val})

    @functools.partial(
        pl.kernel,
        compiler_params=cp,
        out_type=(
            jax.ShapeDtypeStruct((_B,), jnp.int32),
            jax.ShapeDtypeStruct((_B,), jnp.float32),
            jax.ShapeDtypeStruct((_B,), jnp.float32),
        ),
        mesh=mesh,
        scratch_types=[
            pltpu.VMEM((_RPW, _L), jnp.float32),
            pltpu.VMEM((_RPW, _L), jnp.float32),
            pltpu.VMEM((_RPW,), jnp.float32),
            pltpu.VMEM((_RPW,), jnp.float32),
            pltpu.VMEM((_RPW,), jnp.int32),
            pltpu.VMEM((_RPW,), jnp.int32),
            pltpu.VMEM((_RPW,), jnp.int32),
            pltpu.VMEM((_RPW,), jnp.int32),
            pltpu.VMEM((_RPW,), jnp.int32),
            pltpu.VMEM((_RPW,), jnp.int32),
            pltpu.VMEM((_RPW,), jnp.float32),
            pltpu.VMEM((_RPW,), jnp.float32),
            pltpu.SemaphoreType.DMA,
        ],
    )
    def kern(ls_hbm, us_hbm, ip_hbm, s_hbm, nl_hbm, nu_hbm,
             bufa, bufb, pzv, ulv, lov, hiv, midv, idxa, idxb,
             sv, nlv, nuv, sem):
        wid = lax.axis_index("s") * _NC + lax.axis_index("c")
        base = wid * _RPW
        iota = lax.iota(jnp.int32, _L)

        @pl.loop(0, _G)
        def _(g):
            sl = pl.ds(g * _L, _L)
            rows = base + g * _L + iota
            idxa[sl] = rows * (_D // _L)
            idxb[sl] = rows * _BPR + (_BPR - 1)
            lov[sl] = jnp.zeros((_L,), jnp.int32)
            hiv[sl] = jnp.full((_L,), _V, jnp.int32)

        cps = []
        for k in range(_NCHUNK):
            ck = pl.ds(k * _CH, _CH)
            cps.append(pltpu.async_copy(ip_hbm.at[idxa.at[ck]], bufa.at[ck], sem))
            cps.append(pltpu.async_copy(us_hbm.at[idxb.at[ck]], bufb.at[ck], sem))
        for cp in cps:
            cp.wait()

        col_cur = jnp.full((_L,), _CUR, jnp.int32)
        col_last = jnp.full((_L,), _L - 1, jnp.int32)

        @pl.loop(0, _G)
        def _(g):
            sl = pl.ds(g * _L, _L)
            rows16 = g * _L + iota
            pzv[sl] = plsc.load_gather(bufa, [rows16, col_cur])
            ulv[sl] = plsc.load_gather(bufb, [rows16, col_last])

        @pl.loop(0, 10)
        def _(r):
            @pl.loop(0, _G)
            def _(g):
                sl = pl.ds(g * _L, _L)
                mid = (lov[sl] + hiv[sl]) >> 1
                midv[sl] = mid
                idxa[sl] = (base + g * _L + iota) * _BPR + (mid >> 4)

            cps = []
            for k in range(_NCHUNK):
                ck = pl.ds(k * _CH, _CH)
                cps.append(
                    pltpu.async_copy(ls_hbm.at[idxa.at[ck]], bufa.at[ck], sem))
            for cp in cps:
                cp.wait()

            @pl.loop(0, _G)
            def _(g):
                sl = pl.ds(g * _L, _L)
                mid = midv[sl]
                vals = plsc.load_gather(bufa, [g * _L + iota, mid & (_L - 1)])
                less = vals < pzv[sl]
                lov[sl] = jnp.where(less, mid + 1, lov[sl])
                hiv[sl] = jnp.where(less, hiv[sl], mid)

        @pl.loop(0, _G)
        def _(g):
            sl = pl.ds(g * _L, _L)
            s = jnp.maximum(lov[sl] - 1, 0)
            s = jnp.where(pzv[sl] > ulv[sl], 0, s)
            sv[sl] = s
            idxa[sl] = (base + g * _L + iota) * _BPR + (s >> 4)

        cps = []
        for k in range(_NCHUNK):
            ck = pl.ds(k * _CH, _CH)
            cps.append(pltpu.async_copy(ls_hbm.at[idxa.at[ck]], bufa.at[ck], sem))
            cps.append(pltpu.async_copy(us_hbm.at[idxa.at[ck]], bufb.at[ck], sem))
        for cp in cps:
            cp.wait()

        @pl.loop(0, _G)
        def _(g):
            sl = pl.ds(g * _L, _L)
            lane = sv[sl] & (_L - 1)
            rows16 = g * _L + iota
            nlv[sl] = plsc.load_gather(bufa, [rows16, lane])
            nuv[sl] = plsc.load_gather(bufb, [rows16, lane])

        out = pl.ds(base, _RPW)
        pltpu.sync_copy(sv, s_hbm.at[out])
        pltpu.sync_copy(nlv, nl_hbm.at[out])
        pltpu.sync_copy(nuv, nu_hbm.at[out])

    return kern(ls_v, us_v, ip_v)


def _tc_patch(low, upp, nl, nu):
    rows = 1024

    def body(low_ref, upp_ref, nl_ref, nu_ref, outl_ref, outu_ref):
        col = lax.broadcasted_iota(jnp.int32, (rows, _D), 1)
        is_cur = col == _CUR
        outl_ref[...] = jnp.where(is_cur, nl_ref[...], low_ref[...])
        outu_ref[...] = jnp.where(is_cur, nu_ref[...], upp_ref[...])

    return pl.pallas_call(
        body,
        grid=(_B // rows,),
        in_specs=[
            pl.BlockSpec((rows, _D), lambda i: (i, 0)),
            pl.BlockSpec((rows, _D), lambda i: (i, 0)),
            pl.BlockSpec((rows, 1), lambda i: (i, 0)),
            pl.BlockSpec((rows, 1), lambda i: (i, 0)),
        ],
        out_specs=[
            pl.BlockSpec((rows, _D), lambda i: (i, 0)),
            pl.BlockSpec((rows, _D), lambda i: (i, 0)),
        ],
        out_shape=[
            jax.ShapeDtypeStruct((_B, _D), jnp.float32),
            jax.ShapeDtypeStruct((_B, _D), jnp.float32),
        ],
    )(low, upp, nl.reshape(_B, 1), nu.reshape(_B, 1))


def kernel(Ls, Us, low_bound, upp_bound, input_point):
    ls_v = Ls.reshape(_B * _BPR, _L)
    us_v = Us.reshape(_B * _BPR, _L)
    ip_v = input_point.reshape(_B * (_D // _L), _L)
    s, nl, nu = _sc_search(ls_v, us_v, ip_v)
    out_l, out_u = _tc_patch(low_bound, upp_bound, nl, nu)
    return s, out_l, out_u


# speedup vs baseline: 1.6933x; 1.6933x over previous
import dataclasses
import functools

import jax
import jax.numpy as jnp
from jax import lax
from jax.experimental import pallas as pl
from jax.experimental.pallas import tpu as pltpu
from jax.experimental.pallas import tpu_sc as plsc

_B = 16384
_V = 1024
_D = 128
_CUR = 5
_L = 16
_NC = 2
_NS = 16
_NW = _NC * _NS
_RPW = _B // _NW
_SLAB = 16
_NSLAB = _RPW // _SLAB


def _sc_search(ls, pz, ulast):
    mesh = plsc.VectorSubcoreMesh(
        core_axis_name="c", subcore_axis_name="s", num_cores=_NC,
        num_subcores=_NS)
    cp = pltpu.CompilerParams()
    for field, val in (("needs_layout_passes", False),
                       ("use_tc_tiling_on_sc", True)):
        if field in pltpu.CompilerParams.__dataclass_fields__:
            cp = dataclasses.replace(cp, **{field: val})

    @functools.partial(
        pl.kernel,
        compiler_params=cp,
        out_type=(
            jax.ShapeDtypeStruct((_B,), jnp.int32),
            jax.ShapeDtypeStruct((_B,), jnp.float32),
            jax.ShapeDtypeStruct((_B,), jnp.float32),
        ),
        mesh=mesh,
        scratch_types=[
            pltpu.VMEM((_SLAB, _V), jnp.float32),
            pltpu.VMEM((_SLAB, _V), jnp.float32),
            pltpu.VMEM((_RPW,), jnp.float32),
            pltpu.VMEM((_RPW,), jnp.float32),
            pltpu.VMEM((_RPW,), jnp.int32),
            pltpu.VMEM((_RPW,), jnp.float32),
            pltpu.VMEM((_RPW,), jnp.float32),
            pltpu.SemaphoreType.DMA,
            pltpu.SemaphoreType.DMA,
            pltpu.SemaphoreType.DMA,
        ],
    )
    def kern(ls_hbm, pz_hbm, ul_hbm, s_hbm, nl_hbm, nu_hbm,
             bufa, bufb, pzv, ulv, sv, nlv, nuv, sema, semb, semc):
        wid = lax.axis_index("s") * _NC + lax.axis_index("c")
        base = wid * _RPW
        iota = lax.iota(jnp.int32, _L)

        def slab_src(slab):
            return ls_hbm.at[pl.ds(base + slab * _SLAB, _SLAB)]

        pltpu.async_copy(pz_hbm.at[pl.ds(base, _RPW)], pzv, semc)
        pltpu.async_copy(ul_hbm.at[pl.ds(base, _RPW)], ulv, semc)
        pltpu.async_copy(slab_src(0), bufa, sema)
        pltpu.make_async_copy(pz_hbm.at[pl.ds(base, _RPW)], pzv, semc).wait()
        pltpu.make_async_copy(ul_hbm.at[pl.ds(base, _RPW)], ulv, semc).wait()

        def process(slab, buf):
            sl = pl.ds(slab * _L, _L)
            pzg = pzv[sl]
            ulg = ulv[sl]
            lo = jnp.zeros((_L,), jnp.int32)
            hi = jnp.full((_L,), _V, jnp.int32)
            for _ in range(10):
                mid = (lo + hi) >> 1
                vals = plsc.load_gather(buf, [iota, mid])
                less = vals < pzg
                lo = jnp.where(less, mid + 1, lo)
                hi = jnp.where(less, hi, mid)
            s = jnp.maximum(lo - 1, 0)
            s = jnp.where(pzg > ulg, 0, s)
            newl = plsc.load_gather(buf, [iota, s])
            nxt = plsc.load_gather(buf, [iota, jnp.minimum(s + 1, _V - 1)])
            newu = jnp.where(s >= _V - 1, ulg, nxt)
            sv[sl] = s
            nlv[sl] = newl
            nuv[sl] = newu

        @pl.loop(0, _NSLAB // 2)
        def _(k):
            a = 2 * k
            pltpu.make_async_copy(slab_src(a), bufa, sema).wait()
            nb = jnp.minimum(a + 1, _NSLAB - 1)
            pltpu.async_copy(slab_src(nb), bufb, semb)
            process(a, bufa)
            pltpu.make_async_copy(slab_src(nb), bufb, semb).wait()
            na = jnp.minimum(a + 2, _NSLAB - 1)
            pltpu.async_copy(slab_src(na), bufa, sema)
            process(a + 1, bufb)

        pltpu.make_async_copy(slab_src(_NSLAB - 1), bufa, sema).wait()

        out = pl.ds(base, _RPW)
        pltpu.sync_copy(sv, s_hbm.at[out])
        pltpu.sync_copy(nlv, nl_hbm.at[out])
        pltpu.sync_copy(nuv, nu_hbm.at[out])

    return kern(ls, pz, ulast)


def _tc_copy_patch(low, upp, nl, nu):
    rows = 1024

    def body(low_ref, upp_ref, nl_ref, nu_ref, outl_ref, outu_ref):
        col = lax.broadcasted_iota(jnp.int32, (rows, _D), 1)
        is_cur = col == _CUR
        outl_ref[...] = jnp.where(is_cur, nl_ref[...], low_ref[...])
        outu_ref[...] = jnp.where(is_cur, nu_ref[...], upp_ref[...])

    return pl.pallas_call(
        body,
        grid=(_B // rows,),
        in_specs=[
            pl.BlockSpec((rows, _D), lambda i: (i, 0)),
            pl.BlockSpec((rows, _D), lambda i: (i, 0)),
            pl.BlockSpec((rows, 1), lambda i: (i, 0)),
            pl.BlockSpec((rows, 1), lambda i: (i, 0)),
        ],
        out_specs=[
            pl.BlockSpec((rows, _D), lambda i: (i, 0)),
            pl.BlockSpec((rows, _D), lambda i: (i, 0)),
        ],
        out_shape=[
            jax.ShapeDtypeStruct((_B, _D), jnp.float32),
            jax.ShapeDtypeStruct((_B, _D), jnp.float32),
        ],
    )(low, upp, nl.reshape(_B, 1), nu.reshape(_B, 1))


def kernel(Ls, Us, low_bound, upp_bound, input_point):
    pz = input_point[:, _CUR]
    ulast = Us[:, _V - 1]
    s, nl, nu = _sc_search(Ls, pz, ulast)
    outl, outu = _tc_copy_patch(low_bound, upp_bound, nl, nu)
    return s, outl, outu


# speedup vs baseline: 2.6676x; 1.5754x over previous
import dataclasses
import functools

import jax
import jax.numpy as jnp
from jax import lax
from jax.experimental import pallas as pl
from jax.experimental.pallas import tpu as pltpu
from jax.experimental.pallas import tpu_sc as plsc

_B = 16384
_V = 1024
_D = 128
_CUR = 5
_L = 16
_NC = 2
_NS = 16
_NW = _NC * _NS
_RPW = _B // _NW
_SLAB = 16
_NSLAB = _RPW // _SLAB
_NBUF = 4
_UTILE = (_V // _D) - 1


def _sc_all(ls, low, upp, ip, us):
    mesh = plsc.VectorSubcoreMesh(
        core_axis_name="c", subcore_axis_name="s", num_cores=_NC,
        num_subcores=_NS)
    cp = pltpu.CompilerParams()
    for field, val in (("needs_layout_passes", False),
                       ("use_tc_tiling_on_sc", True)):
        if field in pltpu.CompilerParams.__dataclass_fields__:
            cp = dataclasses.replace(cp, **{field: val})

    @functools.partial(
        pl.kernel,
        compiler_params=cp,
        out_type=(
            jax.ShapeDtypeStruct((_B,), jnp.int32),
            jax.ShapeDtypeStruct((_B, _D), jnp.float32),
            jax.ShapeDtypeStruct((_B, _D), jnp.float32),
        ),
        mesh=mesh,
        scratch_types=[
            [pltpu.VMEM((_SLAB, _V), jnp.float32) for _ in range(_NBUF)],
            [pltpu.VMEM((_SLAB, _D), jnp.float32) for _ in range(_NBUF)],
            [pltpu.VMEM((_SLAB, _D), jnp.float32) for _ in range(_NBUF)],
            [pltpu.VMEM((_SLAB, _D), jnp.float32) for _ in range(_NBUF)],
            [pltpu.VMEM((_SLAB, _D), jnp.float32) for _ in range(_NBUF)],
            pltpu.VMEM((_RPW,), jnp.int32),
            [pltpu.SemaphoreType.DMA for _ in range(_NBUF)],
            [pltpu.SemaphoreType.DMA for _ in range(_NBUF)],
        ],
    )
    def kern(ls_hbm, low_hbm, upp_hbm, ip_hbm, us_hbm,
             s_hbm, outl_hbm, outu_hbm,
             lsb, lob, upb, ipb, usb, sv, isems, osems):
        wid = lax.axis_index("s") * _NC + lax.axis_index("c")
        base = wid * _RPW
        iota = lax.iota(jnp.int32, _L)
        col_cur = jnp.full((_L,), _CUR, jnp.int32)
        col_last = jnp.full((_L,), _D - 1, jnp.int32)

        def fills(slab, j):
            r = pl.ds(base + slab * _SLAB, _SLAB)
            ut = pl.ds(_UTILE * _D, _D)
            return [
                pltpu.make_async_copy(ls_hbm.at[r], lsb[j], isems[j]),
                pltpu.make_async_copy(low_hbm.at[r], lob[j], isems[j]),
                pltpu.make_async_copy(upp_hbm.at[r], upb[j], isems[j]),
                pltpu.make_async_copy(ip_hbm.at[r], ipb[j], isems[j]),
                pltpu.make_async_copy(us_hbm.at[r, ut], usb[j], isems[j]),
            ]

        def drains(slab, j):
            r = pl.ds(base + slab * _SLAB, _SLAB)
            return [
                pltpu.make_async_copy(lob[j], outl_hbm.at[r], osems[j]),
                pltpu.make_async_copy(upb[j], outu_hbm.at[r], osems[j]),
            ]

        for c in range(_NBUF - 1):
            for cp_ in fills(c, c):
                cp_.start()

        def step(k, j):
            cur = k * _NBUF + j
            for cp_ in fills(cur, j):
                cp_.wait()
            pzg = plsc.load_gather(ipb[j], [iota, col_cur])
            ulg = plsc.load_gather(usb[j], [iota, col_last])
            lo = jnp.zeros((_L,), jnp.int32)
            hi = jnp.full((_L,), _V, jnp.int32)
            for _ in range(10):
                mid = (lo + hi) >> 1
                vals = plsc.load_gather(lsb[j], [iota, mid])
                less = vals < pzg
                lo = jnp.where(less, mid + 1, lo)
                hi = jnp.where(less, hi, mid)
            s = jnp.maximum(lo - 1, 0)
            s = jnp.where(pzg > ulg, 0, s)
            newl = plsc.load_gather(lsb[j], [iota, s])
            nxtv = plsc.load_gather(lsb[j], [iota, jnp.minimum(s + 1, _V - 1)])
            newu = jnp.where(s >= _V - 1, ulg, nxtv)
            sv[pl.ds(cur * _SLAB, _SLAB)] = s
            plsc.store_scatter(lob[j], [iota, col_cur], newl)
            plsc.store_scatter(upb[j], [iota, col_cur], newu)
            for cp_ in drains(cur, j):
                cp_.start()
            j2 = (j + _NBUF - 1) % _NBUF
            nxt = jnp.minimum(cur + _NBUF - 1, _NSLAB - 1)
            prev = cur - 1

            def waits_and_fill():
                for cp_ in drains(prev, j2):
                    cp_.wait()
                for cp_ in fills(nxt, j2):
                    cp_.start()

            if j == 0:
                @pl.when(k >= 1)
                def _():
                    waits_and_fill()

                @pl.when(k < 1)
                def _():
                    for cp_ in fills(nxt, j2):
                        cp_.start()
            else:
                waits_and_fill()

        @pl.loop(0, _NSLAB // _NBUF)
        def _(k):
            for j in range(_NBUF):
                step(k, j)

        for j in range(_NBUF - 1):
            for cp_ in fills(_NSLAB - 1, j):
                cp_.wait()
        for cp_ in drains(_NSLAB - 1, _NBUF - 1):
            cp_.wait()

        pltpu.sync_copy(sv, s_hbm.at[pl.ds(base, _RPW)])

    return kern(ls, low, upp, ip, us)


def kernel(Ls, Us, low_bound, upp_bound, input_point):
    return _sc_all(Ls, low_bound, upp_bound, input_point, Us)
